# baseline (speedup 1.0000x reference)
import functools

import jax
import jax.numpy as jnp
from jax import lax
from jax.experimental import pallas as pl
from jax.experimental.pallas import tpu as pltpu
from jax.experimental.pallas import tpu_sc as plsc

BATCH = 16384
DIM = 32
V = 1000000
_NC = 2
_NS = 16
_NW = _NC * _NS
_BPW = BATCH // _NW
NCHUNK_FULL = V // 128
TAIL = V - NCHUNK_FULL * 128
ROWS = V // 4
CPW = (NCHUNK_FULL + _NW - 1) // _NW

_mesh = plsc.VectorSubcoreMesh(core_axis_name="c", subcore_axis_name="s")
_params = pltpu.CompilerParams(
    use_tc_tiling_on_sc=True, needs_layout_passes=False)

def _lane():
  return lax.broadcasted_iota(jnp.int32, (16,), 0)


def _transpose_chunk(buf, lin_v, n_ent):
  lane = _lane()
  d_lo = lane
  d_hi = lane + 16

  def body(l, _):
    r = jnp.broadcast_to((l * 32) // 128, (16,))
    c0 = (l * 32) % 128
    lv = jnp.broadcast_to(l, (16,))
    v0 = plsc.load_gather(buf, [d_lo, lv])
    plsc.store_scatter(lin_v, [r, lane + c0], v0)
    v1 = plsc.load_gather(buf, [d_hi, lv])
    plsc.store_scatter(lin_v, [r, lane + (c0 + 16)], v1)
    return 0

  lax.fori_loop(0, n_ent, body, 0, unroll=4)


@functools.partial(
    pl.kernel,
    mesh=_mesh,
    out_type=[
        jax.ShapeDtypeStruct((ROWS, 128), jnp.float32),
        jax.ShapeDtypeStruct((ROWS, 128), jnp.float32),
    ],
    scratch_types=[
        pltpu.VMEM((32, 128), jnp.float32),
        pltpu.VMEM((32, 128), jnp.float32),
        pltpu.VMEM((32, 128), jnp.float32),
        pltpu.SemaphoreType.DMA,
        pltpu.SemaphoreType.DMA,
    ],
    compiler_params=_params,
)
def _relayout(ent_t, rel_t, ent_tail, rel_tail, ent_lin, rel_lin,
              buf0, buf1, lin_v, sem0, sem1):
  w = lax.axis_index("s") * _NC + lax.axis_index("c")

  for tab, tail_blk, lin in (
      (ent_t, ent_tail, ent_lin),
      (rel_t, rel_tail, rel_lin),
  ):
    def start(j, buf, sem):
      tc = j * _NW + w

      @pl.when(tc < NCHUNK_FULL)
      def _():
        off = pl.multiple_of(tc * 128, 128)
        pltpu.async_copy(tab.at[:, pl.ds(off, 128)], buf, sem)

    def finish(j, buf, sem):
      tc = j * _NW + w

      @pl.when(tc < NCHUNK_FULL)
      def _():
        pltpu.make_async_copy(tab.at[:, pl.ds(0, 128)], buf, sem).wait()
        _transpose_chunk(buf, lin_v, 128)
        roff = pl.multiple_of(tc * 32, 8)
        pltpu.sync_copy(lin_v, lin.at[pl.ds(roff, 32), :])

    start(0, buf0, sem0)

    def body(j, _):
      buf, sem = buf0, sem0
      alt, asem = buf1, sem1
      @pl.when(j % 2 == 0)
      def _():
        start(j + 1, buf1, sem1)
        finish(j, buf0, sem0)

      @pl.when(j % 2 == 1)
      def _():
        start(j + 1, buf0, sem0)
        finish(j, buf1, sem1)

      return 0

    lax.fori_loop(0, CPW, body, 0, unroll=False)

    @pl.when(w == NCHUNK_FULL % _NW)
    def _():
      pltpu.sync_copy(tail_blk, buf0)
      _transpose_chunk(buf0, lin_v, TAIL)
      pltpu.sync_copy(
          lin_v.at[pl.ds(0, TAIL * 32 // 128), :],
          lin.at[pl.ds(pl.multiple_of(NCHUNK_FULL * 32, 8), TAIL * 32 // 128), :])


@functools.partial(
    pl.kernel,
    mesh=_mesh,
    out_type=[
        jax.ShapeDtypeStruct((32, BATCH), jnp.float32),
        jax.ShapeDtypeStruct((32, BATCH), jnp.float32),
        jax.ShapeDtypeStruct((32, BATCH), jnp.float32),
    ],
    scratch_types=[
        pltpu.VMEM((_BPW,), jnp.int32),
        pltpu.VMEM((_BPW,), jnp.int32),
        pltpu.VMEM((_BPW, 128), jnp.float32),
        pltpu.VMEM((32, _BPW), jnp.float32),
        pltpu.SemaphoreType.DMA,
    ],
    compiler_params=_params,
)
def _gather3(head_h, rel_h, tail_h, ent_lin, rel_lin,
             out_h, out_r, out_t,
             idx_v, row_v, rows_v, stage_v, sem):
  w = lax.axis_index("s") * _NC + lax.axis_index("c")
  base = w * _BPW
  sl = pl.ds(pl.multiple_of(base, 128), _BPW)
  lane = _lane()

  for idx_hbm, lin, out in (
      (head_h, ent_lin, out_h),
      (rel_h, rel_lin, out_r),
      (tail_h, ent_lin, out_t),
  ):
    pltpu.sync_copy(idx_hbm.at[sl], idx_v)

    def rowids(b, _):
      bb = jnp.broadcast_to(b * 16, (16,)) + lane
      e = plsc.load_gather(idx_v, [bb])
      plsc.store_scatter(row_v, [bb], e >> 2)
      return 0

    lax.fori_loop(0, _BPW // 16, rowids, 0, unroll=4)
    pltpu.async_copy(lin.at[row_v], rows_v, sem).wait()

    def extract(b, _):
      bb = jnp.broadcast_to(b * 16, (16,)) + lane
      e = plsc.load_gather(idx_v, [bb])
      col = (e & 3) * 32
      for d in range(32):
        dv = jnp.broadcast_to(d, (16,)).astype(jnp.int32)
        v = plsc.load_gather(rows_v, [bb, col + d])
        plsc.store_scatter(stage_v, [dv, bb], v)
      return 0

    lax.fori_loop(0, _BPW // 16, extract, 0, unroll=False)
    pltpu.sync_copy(stage_v, out.at[:, sl])


def kernel(raw_triples, entity_embeddings, relation_embeddings):
  tri = raw_triples.astype(jnp.int32)
  head = tri[:, 0]
  rel = tri[:, 1]
  tail = tri[:, 2]
  ent_tail = jnp.pad(entity_embeddings.T[:, NCHUNK_FULL * 128:],
                     ((0, 0), (0, 128 - TAIL)))
  rel_tail = jnp.pad(relation_embeddings.T[:, NCHUNK_FULL * 128:],
                     ((0, 0), (0, 128 - TAIL)))
  ent_lin, rel_lin = _relayout(entity_embeddings.T, relation_embeddings.T,
                               ent_tail, rel_tail)
  out_h, out_r, out_t = _gather3(head, rel, tail, ent_lin, rel_lin)
  return (out_h.T, out_r.T, out_t.T)


# speedup vs baseline: 1.2009x; 1.2009x over previous
import functools

import jax
import jax.numpy as jnp
from jax import lax
from jax.experimental import pallas as pl
from jax.experimental.pallas import tpu as pltpu
from jax.experimental.pallas import tpu_sc as plsc

BATCH = 16384
DIM = 32
V = 1000000
_NC = 2
_NS = 16
_NW = _NC * _NS
_BPW = BATCH // _NW
NCHUNK_FULL = V // 128
TAIL = V - NCHUNK_FULL * 128
ROWS = V // 4
CPW = (NCHUNK_FULL + _NW - 1) // _NW

_mesh = plsc.VectorSubcoreMesh(core_axis_name="c", subcore_axis_name="s")
_params = pltpu.CompilerParams(
    use_tc_tiling_on_sc=True, needs_layout_passes=False)

def _lane():
  return lax.broadcasted_iota(jnp.int32, (16,), 0)


def _transpose_chunk(buf, lin_v, n_ent):
  lane = _lane()
  rbase = lane >> 2
  cbase = (lane & 3) * 32

  def body(k, _):
    m = k >> 2
    g = (k & 3) * 8
    lv = lane + m * 16
    rv = rbase + m * 4
    for dd in range(8):
      d = g + dd
      v = plsc.load_gather(buf, [jnp.broadcast_to(d, (16,)).astype(jnp.int32), lv])
      plsc.store_scatter(lin_v, [rv, cbase + d], v)
    return 0

  lax.fori_loop(0, (n_ent // 16) * 4, body, 0, unroll=2)


@functools.partial(
    pl.kernel,
    mesh=_mesh,
    out_type=[
        jax.ShapeDtypeStruct((ROWS, 128), jnp.float32),
        jax.ShapeDtypeStruct((ROWS, 128), jnp.float32),
    ],
    scratch_types=[
        pltpu.VMEM((32, 128), jnp.float32),
        pltpu.VMEM((32, 128), jnp.float32),
        pltpu.VMEM((32, 128), jnp.float32),
        pltpu.SemaphoreType.DMA,
        pltpu.SemaphoreType.DMA,
    ],
    compiler_params=_params,
)
def _relayout(ent_t, rel_t, ent_tail, rel_tail, ent_lin, rel_lin,
              buf0, buf1, lin_v, sem0, sem1):
  w = lax.axis_index("s") * _NC + lax.axis_index("c")

  for tab, tail_blk, lin in (
      (ent_t, ent_tail, ent_lin),
      (rel_t, rel_tail, rel_lin),
  ):
    def start(j, buf, sem):
      tc = j * _NW + w

      @pl.when(tc < NCHUNK_FULL)
      def _():
        off = pl.multiple_of(tc * 128, 128)
        pltpu.async_copy(tab.at[:, pl.ds(off, 128)], buf, sem)

    def finish(j, buf, sem):
      tc = j * _NW + w

      @pl.when(tc < NCHUNK_FULL)
      def _():
        pltpu.make_async_copy(tab.at[:, pl.ds(0, 128)], buf, sem).wait()
        _transpose_chunk(buf, lin_v, 128)
        roff = pl.multiple_of(tc * 32, 8)
        pltpu.sync_copy(lin_v, lin.at[pl.ds(roff, 32), :])

    start(0, buf0, sem0)

    def body(j, _):
      buf, sem = buf0, sem0
      alt, asem = buf1, sem1
      @pl.when(j % 2 == 0)
      def _():
        start(j + 1, buf1, sem1)
        finish(j, buf0, sem0)

      @pl.when(j % 2 == 1)
      def _():
        start(j + 1, buf0, sem0)
        finish(j, buf1, sem1)

      return 0

    lax.fori_loop(0, CPW, body, 0, unroll=False)

    @pl.when(w == NCHUNK_FULL % _NW)
    def _():
      pltpu.sync_copy(tail_blk, buf0)
      _transpose_chunk(buf0, lin_v, TAIL)
      pltpu.sync_copy(
          lin_v.at[pl.ds(0, TAIL * 32 // 128), :],
          lin.at[pl.ds(pl.multiple_of(NCHUNK_FULL * 32, 8), TAIL * 32 // 128), :])


@functools.partial(
    pl.kernel,
    mesh=_mesh,
    out_type=[
        jax.ShapeDtypeStruct((32, BATCH), jnp.float32),
        jax.ShapeDtypeStruct((32, BATCH), jnp.float32),
        jax.ShapeDtypeStruct((32, BATCH), jnp.float32),
    ],
    scratch_types=[
        pltpu.VMEM((_BPW,), jnp.int32),
        pltpu.VMEM((_BPW,), jnp.int32),
        pltpu.VMEM((_BPW, 128), jnp.float32),
        pltpu.VMEM((32, _BPW), jnp.float32),
        pltpu.SemaphoreType.DMA,
    ],
    compiler_params=_params,
)
def _gather3(head_h, rel_h, tail_h, ent_lin, rel_lin,
             out_h, out_r, out_t,
             idx_v, row_v, rows_v, stage_v, sem):
  w = lax.axis_index("s") * _NC + lax.axis_index("c")
  base = w * _BPW
  sl = pl.ds(pl.multiple_of(base, 128), _BPW)
  lane = _lane()

  for idx_hbm, lin, out in (
      (head_h, ent_lin, out_h),
      (rel_h, rel_lin, out_r),
      (tail_h, ent_lin, out_t),
  ):
    pltpu.sync_copy(idx_hbm.at[sl], idx_v)

    def rowids(b, _):
      bb = jnp.broadcast_to(b * 16, (16,)) + lane
      e = plsc.load_gather(idx_v, [bb])
      plsc.store_scatter(row_v, [bb], e >> 2)
      return 0

    lax.fori_loop(0, _BPW // 16, rowids, 0, unroll=4)
    pltpu.async_copy(lin.at[row_v], rows_v, sem).wait()

    def extract(b, _):
      bb = jnp.broadcast_to(b * 16, (16,)) + lane
      e = plsc.load_gather(idx_v, [bb])
      col = (e & 3) * 32
      for d in range(32):
        dv = jnp.broadcast_to(d, (16,)).astype(jnp.int32)
        v = plsc.load_gather(rows_v, [bb, col + d])
        plsc.store_scatter(stage_v, [dv, bb], v)
      return 0

    lax.fori_loop(0, _BPW // 16, extract, 0, unroll=False)
    pltpu.sync_copy(stage_v, out.at[:, sl])


def kernel(raw_triples, entity_embeddings, relation_embeddings):
  tri = raw_triples.astype(jnp.int32)
  head = tri[:, 0]
  rel = tri[:, 1]
  tail = tri[:, 2]
  ent_tail = jnp.pad(entity_embeddings.T[:, NCHUNK_FULL * 128:],
                     ((0, 0), (0, 128 - TAIL)))
  rel_tail = jnp.pad(relation_embeddings.T[:, NCHUNK_FULL * 128:],
                     ((0, 0), (0, 128 - TAIL)))
  ent_lin, rel_lin = _relayout(entity_embeddings.T, relation_embeddings.T,
                               ent_tail, rel_tail)
  out_h, out_r, out_t = _gather3(head, rel, tail, ent_lin, rel_lin)
  return (out_h.T, out_r.T, out_t.T)


# speedup vs baseline: 1.5817x; 1.3171x over previous
import functools

import jax
import jax.numpy as jnp
from jax import lax
from jax.experimental import pallas as pl
from jax.experimental.pallas import tpu as pltpu
from jax.experimental.pallas import tpu_sc as plsc

BATCH = 16384
DIM = 32
_NC = 2
_NS = 16
_NW = _NC * _NS
_BPW = BATCH // _NW

_mesh = plsc.VectorSubcoreMesh(core_axis_name="c", subcore_axis_name="s")


@functools.partial(
    pl.kernel,
    mesh=_mesh,
    out_type=[
        jax.ShapeDtypeStruct((BATCH, DIM), jnp.bfloat16),
        jax.ShapeDtypeStruct((BATCH, DIM), jnp.bfloat16),
        jax.ShapeDtypeStruct((BATCH, DIM), jnp.bfloat16),
    ],
    scratch_types=[
        pltpu.VMEM((_BPW,), jnp.int32),
        pltpu.VMEM((_BPW,), jnp.int32),
        pltpu.VMEM((_BPW,), jnp.int32),
        pltpu.VMEM((_BPW, DIM), jnp.bfloat16),
        pltpu.VMEM((_BPW, DIM), jnp.bfloat16),
        pltpu.VMEM((_BPW, DIM), jnp.bfloat16),
        pltpu.SemaphoreType.DMA,
    ],
    compiler_params=pltpu.CompilerParams(use_tc_tiling_on_sc=False),
)
def _gather3(head_hbm, rel_hbm, tail_hbm, ent_hbm, reltab_hbm,
             out_h, out_r, out_t,
             idx_h, idx_r, idx_t, rows_h, rows_r, rows_t, sem):
    wid = lax.axis_index("s") * _NC + lax.axis_index("c")
    base = wid * _BPW
    sl = pl.ds(base, _BPW)
    pltpu.sync_copy(head_hbm.at[sl], idx_h)
    pltpu.sync_copy(rel_hbm.at[sl], idx_r)
    pltpu.sync_copy(tail_hbm.at[sl], idx_t)
    ch = pltpu.async_copy(ent_hbm.at[idx_h], rows_h, sem)
    cr = pltpu.async_copy(reltab_hbm.at[idx_r], rows_r, sem)
    ct = pltpu.async_copy(ent_hbm.at[idx_t], rows_t, sem)
    ch.wait()
    cr.wait()
    ct.wait()
    pltpu.sync_copy(rows_h, out_h.at[sl])
    pltpu.sync_copy(rows_r, out_r.at[sl])
    pltpu.sync_copy(rows_t, out_t.at[sl])


def kernel(raw_triples, entity_embeddings, relation_embeddings):
    tri = raw_triples.astype(jnp.int32)
    head = tri[:, 0]
    rel = tri[:, 1]
    tail = tri[:, 2]
    ent16 = entity_embeddings.astype(jnp.bfloat16)
    rel16 = relation_embeddings.astype(jnp.bfloat16)
    head_emb, rel_emb, tail_emb = _gather3(head, rel, tail, ent16, rel16)
    return (head_emb.astype(jnp.float32),
            rel_emb.astype(jnp.float32),
            tail_emb.astype(jnp.float32))


# speedup vs baseline: 1.8263x; 1.1547x over previous
import functools

import jax
import jax.numpy as jnp
from jax import lax
from jax.experimental import pallas as pl
from jax.experimental.pallas import tpu as pltpu
from jax.experimental.pallas import tpu_sc as plsc

BATCH = 16384
DIM = 32
_NC = 2
_NS = 16
_NW = _NC * _NS
_BPW = BATCH // _NW

_mesh = plsc.VectorSubcoreMesh(core_axis_name="c", subcore_axis_name="s")


@functools.partial(
    pl.kernel,
    mesh=_mesh,
    out_type=[
        jax.ShapeDtypeStruct((BATCH, DIM), jnp.float32),
        jax.ShapeDtypeStruct((BATCH, DIM), jnp.float32),
        jax.ShapeDtypeStruct((BATCH, DIM), jnp.float32),
    ],
    scratch_types=[
        pltpu.VMEM((_BPW,), jnp.int32),
        pltpu.VMEM((_BPW,), jnp.int32),
        pltpu.VMEM((_BPW,), jnp.int32),
        pltpu.VMEM((_BPW, DIM), jnp.float32),
        pltpu.VMEM((_BPW, DIM), jnp.float32),
        pltpu.VMEM((_BPW, DIM), jnp.float32),
        pltpu.SemaphoreType.DMA,
    ],
    compiler_params=pltpu.CompilerParams(use_tc_tiling_on_sc=False),
)
def _gather3(head_hbm, rel_hbm, tail_hbm, ent_hbm, reltab_hbm,
             out_h, out_r, out_t,
             idx_h, idx_r, idx_t, rows_h, rows_r, rows_t, sem):
    wid = lax.axis_index("s") * _NC + lax.axis_index("c")
    base = wid * _BPW
    sl = pl.ds(base, _BPW)
    pltpu.sync_copy(head_hbm.at[sl], idx_h)
    pltpu.sync_copy(rel_hbm.at[sl], idx_r)
    pltpu.sync_copy(tail_hbm.at[sl], idx_t)
    ch = pltpu.async_copy(ent_hbm.at[idx_h], rows_h, sem)
    cr = pltpu.async_copy(reltab_hbm.at[idx_r], rows_r, sem)
    ct = pltpu.async_copy(ent_hbm.at[idx_t], rows_t, sem)
    ch.wait()
    cr.wait()
    ct.wait()
    pltpu.sync_copy(rows_h, out_h.at[sl])
    pltpu.sync_copy(rows_r, out_r.at[sl])
    pltpu.sync_copy(rows_t, out_t.at[sl])


def kernel(raw_triples, entity_embeddings, relation_embeddings):
    tri = raw_triples.astype(jnp.int32)
    head = tri[:, 0]
    rel = tri[:, 1]
    tail = tri[:, 2]
    head_emb, rel_emb, tail_emb = _gather3(
        head, rel, tail, entity_embeddings, relation_embeddings)
    return (head_emb, rel_emb, tail_emb)
